# baseline (speedup 1.0000x reference)
import jax
import jax.numpy as jnp
from jax.experimental import pallas as pl
from jax.experimental.pallas import tpu as pltpu

_BM1 = 400
_BM2 = 1000


_F8 = jnp.float8_e4m3fn
_F4 = jnp.float4_e2m1fn
_LSCALE = 128.0
_HSCALE = 2.0
_RSCALE = 32.0


def _layer1_kernel(l_ref, x16_ref, xs_ref, w1_ref, w1s_ref, b1_ref,
                   l4_ref, r8_ref, h32_ref, h4_ref):
    l = l_ref[...]
    y = l * _LSCALE
    l4 = y.astype(_F4)
    l4_ref[...] = l4
    r8_ref[...] = ((y - l4.astype(jnp.float32)) * _RSCALE).astype(_F8)
    t = jnp.dot(l.astype(jnp.bfloat16), x16_ref[...],
                preferred_element_type=jnp.float32)
    e = jnp.dot(t, w1s_ref[...], preferred_element_type=jnp.float32)
    v = jnp.dot(xs_ref[...], w1_ref[...],
                preferred_element_type=jnp.float32) + b1_ref[...]
    h = jnp.maximum(v - e, 0.0)
    h32_ref[...] = h
    h4_ref[...] = (h * _HSCALE).astype(_F4)


def _hidden_kernel(l4_ref, h4_ref, h32_ref, d_ref, o32_ref, o4_ref):
    e = jnp.dot(l4_ref[...], h4_ref[...], preferred_element_type=jnp.float32)
    h = jnp.maximum(h32_ref[...] - e * d_ref[...], 0.0)
    o32_ref[...] = h
    o4_ref[...] = (h * _HSCALE).astype(_F4)


def _hidden_last_kernel(l4_ref, h4_ref, h32_ref, d_ref, w2_ref, w2s_ref,
                        b2_ref, u2_ref, v2_ref):
    e = jnp.dot(l4_ref[...], h4_ref[...], preferred_element_type=jnp.float32)
    h = jnp.maximum(h32_ref[...] - e * d_ref[...], 0.0)
    u2_ref[...] = jnp.dot(h, w2s_ref[...],
                          preferred_element_type=jnp.float32).astype(jnp.bfloat16)
    v2_ref[...] = jnp.dot(h, w2_ref[...],
                          preferred_element_type=jnp.float32) + b2_ref[...]


def _final_kernel(l4_ref, r8_ref, u2_ref, v2_ref, out_ref):
    e_hi = jnp.dot(l4_ref[...], u2_ref[...], preferred_element_type=jnp.float32)
    e_lo = jnp.dot(r8_ref[...], u2_ref[...], preferred_element_type=jnp.float32)
    e = (e_hi + e_lo * (1.0 / _RSCALE)) * (1.0 / _LSCALE)
    logits = jnp.maximum(v2_ref[...] - e, 0.0)
    m = jnp.max(logits, axis=1, keepdims=True)
    lse = jnp.log(jnp.sum(jnp.exp(logits - m), axis=1, keepdims=True)) + m
    out_ref[...] = logits - lse


def kernel(x, l_sym, W1, b1, d1, dh, W2, b2, d2):
    n, nfeat = x.shape
    nhid = W1.shape[1]
    nclass = W2.shape[1]
    f32 = jnp.float32
    bf16 = jnp.bfloat16
    par = pltpu.CompilerParams(dimension_semantics=("parallel",))

    W1s = ((d1 + 1.0)[:, None] * W1).astype(bf16)
    W2s = (d2 + 1.0)[:, None] * W2
    b1r = b1.reshape(1, nhid)
    b2r = b2.reshape(1, nclass)
    x16 = x.astype(bf16)

    l4, r8, h32, h4 = pl.pallas_call(
        _layer1_kernel,
        grid=(n // _BM1,),
        in_specs=[
            pl.BlockSpec((_BM1, n), lambda i: (i, 0)),
            pl.BlockSpec((n, nfeat), lambda i: (0, 0)),
            pl.BlockSpec((_BM1, nfeat), lambda i: (i, 0)),
            pl.BlockSpec((nfeat, nhid), lambda i: (0, 0)),
            pl.BlockSpec((nfeat, nhid), lambda i: (0, 0)),
            pl.BlockSpec((1, nhid), lambda i: (0, 0)),
        ],
        out_specs=[
            pl.BlockSpec((_BM1, n), lambda i: (i, 0)),
            pl.BlockSpec((_BM1, n), lambda i: (i, 0)),
            pl.BlockSpec((_BM1, nhid), lambda i: (i, 0)),
            pl.BlockSpec((_BM1, nhid), lambda i: (i, 0)),
        ],
        out_shape=[
            jax.ShapeDtypeStruct((n, n), _F4),
            jax.ShapeDtypeStruct((n, n), _F8),
            jax.ShapeDtypeStruct((n, nhid), f32),
            jax.ShapeDtypeStruct((n, nhid), _F4),
        ],
        compiler_params=par,
    )(l_sym, x16, x, W1, W1s, b1r)

    nlayer_hidden = dh.shape[0]
    for i in range(nlayer_hidden - 1):
        dr = (dh[i] / (_LSCALE * _HSCALE)).reshape(1, nhid)
        h32, h4 = pl.pallas_call(
            _hidden_kernel,
            grid=(n // _BM2,),
            in_specs=[
                pl.BlockSpec((_BM2, n), lambda i: (i, 0)),
                pl.BlockSpec((n, nhid), lambda i: (0, 0)),
                pl.BlockSpec((_BM2, nhid), lambda i: (i, 0)),
                pl.BlockSpec((1, nhid), lambda i: (0, 0)),
            ],
            out_specs=[
                pl.BlockSpec((_BM2, nhid), lambda i: (i, 0)),
                pl.BlockSpec((_BM2, nhid), lambda i: (i, 0)),
            ],
            out_shape=[
                jax.ShapeDtypeStruct((n, nhid), f32),
                jax.ShapeDtypeStruct((n, nhid), _F4),
            ],
            compiler_params=par,
        )(l4, h4, h32, dr)

    dr = (dh[nlayer_hidden - 1] / (_LSCALE * _HSCALE)).reshape(1, nhid)
    u2, v2 = pl.pallas_call(
        _hidden_last_kernel,
        grid=(n // _BM2,),
        in_specs=[
            pl.BlockSpec((_BM2, n), lambda i: (i, 0)),
            pl.BlockSpec((n, nhid), lambda i: (0, 0)),
            pl.BlockSpec((_BM2, nhid), lambda i: (i, 0)),
            pl.BlockSpec((1, nhid), lambda i: (0, 0)),
            pl.BlockSpec((nhid, nclass), lambda i: (0, 0)),
            pl.BlockSpec((nhid, nclass), lambda i: (0, 0)),
            pl.BlockSpec((1, nclass), lambda i: (0, 0)),
        ],
        out_specs=[
            pl.BlockSpec((_BM2, nclass), lambda i: (i, 0)),
            pl.BlockSpec((_BM2, nclass), lambda i: (i, 0)),
        ],
        out_shape=[
            jax.ShapeDtypeStruct((n, nclass), bf16),
            jax.ShapeDtypeStruct((n, nclass), f32),
        ],
        compiler_params=par,
    )(l4, h4, h32, dr, W2, W2s, b2r)

    out = pl.pallas_call(
        _final_kernel,
        grid=(n // _BM2,),
        in_specs=[
            pl.BlockSpec((_BM2, n), lambda i: (i, 0)),
            pl.BlockSpec((_BM2, n), lambda i: (i, 0)),
            pl.BlockSpec((n, nclass), lambda i: (0, 0)),
            pl.BlockSpec((_BM2, nclass), lambda i: (i, 0)),
        ],
        out_specs=pl.BlockSpec((_BM2, nclass), lambda i: (i, 0)),
        out_shape=jax.ShapeDtypeStruct((n, nclass), f32),
        compiler_params=par,
    )(l4, r8, u2, v2)

    return out


# speedup vs baseline: 1.0655x; 1.0655x over previous
import jax
import jax.numpy as jnp
from jax.experimental import pallas as pl
from jax.experimental.pallas import tpu as pltpu

_BM1 = 400
_BM2 = 1000


_F8 = jnp.float8_e4m3fn
_F4 = jnp.float4_e2m1fn
_LSCALE = 128.0
_HSCALE = 2.0


def _layer1_kernel(l_ref, x16_ref, xs_ref, w1_ref, w1s_ref, b1_ref,
                   l16_ref, l4_ref, h32_ref, h4_ref):
    l = l_ref[...]
    l16 = l.astype(jnp.bfloat16)
    l16_ref[...] = l16
    l4_ref[...] = (l * _LSCALE).astype(_F4)
    t = jnp.dot(l16, x16_ref[...], preferred_element_type=jnp.float32)
    e = jnp.dot(t, w1s_ref[...], preferred_element_type=jnp.float32)
    v = jnp.dot(xs_ref[...], w1_ref[...],
                preferred_element_type=jnp.float32) + b1_ref[...]
    h = jnp.maximum(v - e, 0.0)
    h32_ref[...] = h
    h4_ref[...] = (h * _HSCALE).astype(_F4)


def _hidden_kernel(l4_ref, h4_ref, h32_ref, d_ref, o32_ref, o4_ref):
    e = jnp.dot(l4_ref[...], h4_ref[...], preferred_element_type=jnp.float32)
    h = jnp.maximum(h32_ref[...] - e * d_ref[...], 0.0)
    o32_ref[...] = h
    o4_ref[...] = (h * _HSCALE).astype(_F4)


def _hidden_last_kernel(l4_ref, h4_ref, h32_ref, d_ref, w2_ref, w2s_ref,
                        b2_ref, u2_ref, v2_ref):
    e = jnp.dot(l4_ref[...], h4_ref[...], preferred_element_type=jnp.float32)
    h = jnp.maximum(h32_ref[...] - e * d_ref[...], 0.0)
    u2_ref[...] = jnp.dot(h, w2s_ref[...],
                          preferred_element_type=jnp.float32).astype(jnp.bfloat16)
    v2_ref[...] = jnp.dot(h, w2_ref[...],
                          preferred_element_type=jnp.float32) + b2_ref[...]


def _final_kernel(l16_ref, u2_ref, v2_ref, out_ref):
    e = jnp.dot(l16_ref[...], u2_ref[...], preferred_element_type=jnp.float32)
    logits = jnp.maximum(v2_ref[...] - e, 0.0)
    m = jnp.max(logits, axis=1, keepdims=True)
    lse = jnp.log(jnp.sum(jnp.exp(logits - m), axis=1, keepdims=True)) + m
    out_ref[...] = logits - lse


def kernel(x, l_sym, W1, b1, d1, dh, W2, b2, d2):
    n, nfeat = x.shape
    nhid = W1.shape[1]
    nclass = W2.shape[1]
    f32 = jnp.float32
    bf16 = jnp.bfloat16
    par = pltpu.CompilerParams(dimension_semantics=("parallel",))

    W1s = ((d1 + 1.0)[:, None] * W1).astype(bf16)
    W2s = (d2 + 1.0)[:, None] * W2
    b1r = b1.reshape(1, nhid)
    b2r = b2.reshape(1, nclass)
    x16 = x.astype(bf16)

    l16, l4, h32, h4 = pl.pallas_call(
        _layer1_kernel,
        grid=(n // _BM1,),
        in_specs=[
            pl.BlockSpec((_BM1, n), lambda i: (i, 0)),
            pl.BlockSpec((n, nfeat), lambda i: (0, 0)),
            pl.BlockSpec((_BM1, nfeat), lambda i: (i, 0)),
            pl.BlockSpec((nfeat, nhid), lambda i: (0, 0)),
            pl.BlockSpec((nfeat, nhid), lambda i: (0, 0)),
            pl.BlockSpec((1, nhid), lambda i: (0, 0)),
        ],
        out_specs=[
            pl.BlockSpec((_BM1, n), lambda i: (i, 0)),
            pl.BlockSpec((_BM1, n), lambda i: (i, 0)),
            pl.BlockSpec((_BM1, nhid), lambda i: (i, 0)),
            pl.BlockSpec((_BM1, nhid), lambda i: (i, 0)),
        ],
        out_shape=[
            jax.ShapeDtypeStruct((n, n), bf16),
            jax.ShapeDtypeStruct((n, n), _F4),
            jax.ShapeDtypeStruct((n, nhid), f32),
            jax.ShapeDtypeStruct((n, nhid), _F4),
        ],
        compiler_params=par,
    )(l_sym, x16, x, W1, W1s, b1r)

    nlayer_hidden = dh.shape[0]
    for i in range(nlayer_hidden - 1):
        dr = (dh[i] / (_LSCALE * _HSCALE)).reshape(1, nhid)
        h32, h4 = pl.pallas_call(
            _hidden_kernel,
            grid=(n // _BM2,),
            in_specs=[
                pl.BlockSpec((_BM2, n), lambda i: (i, 0)),
                pl.BlockSpec((n, nhid), lambda i: (0, 0)),
                pl.BlockSpec((_BM2, nhid), lambda i: (i, 0)),
                pl.BlockSpec((1, nhid), lambda i: (0, 0)),
            ],
            out_specs=[
                pl.BlockSpec((_BM2, nhid), lambda i: (i, 0)),
                pl.BlockSpec((_BM2, nhid), lambda i: (i, 0)),
            ],
            out_shape=[
                jax.ShapeDtypeStruct((n, nhid), f32),
                jax.ShapeDtypeStruct((n, nhid), _F4),
            ],
            compiler_params=par,
        )(l4, h4, h32, dr)

    dr = (dh[nlayer_hidden - 1] / (_LSCALE * _HSCALE)).reshape(1, nhid)
    u2, v2 = pl.pallas_call(
        _hidden_last_kernel,
        grid=(n // _BM2,),
        in_specs=[
            pl.BlockSpec((_BM2, n), lambda i: (i, 0)),
            pl.BlockSpec((n, nhid), lambda i: (0, 0)),
            pl.BlockSpec((_BM2, nhid), lambda i: (i, 0)),
            pl.BlockSpec((1, nhid), lambda i: (0, 0)),
            pl.BlockSpec((nhid, nclass), lambda i: (0, 0)),
            pl.BlockSpec((nhid, nclass), lambda i: (0, 0)),
            pl.BlockSpec((1, nclass), lambda i: (0, 0)),
        ],
        out_specs=[
            pl.BlockSpec((_BM2, nclass), lambda i: (i, 0)),
            pl.BlockSpec((_BM2, nclass), lambda i: (i, 0)),
        ],
        out_shape=[
            jax.ShapeDtypeStruct((n, nclass), bf16),
            jax.ShapeDtypeStruct((n, nclass), f32),
        ],
        compiler_params=par,
    )(l4, h4, h32, dr, W2, W2s, b2r)

    out = pl.pallas_call(
        _final_kernel,
        grid=(n // _BM2,),
        in_specs=[
            pl.BlockSpec((_BM2, n), lambda i: (i, 0)),
            pl.BlockSpec((n, nclass), lambda i: (0, 0)),
            pl.BlockSpec((_BM2, nclass), lambda i: (i, 0)),
        ],
        out_specs=pl.BlockSpec((_BM2, nclass), lambda i: (i, 0)),
        out_shape=jax.ShapeDtypeStruct((n, nclass), f32),
        compiler_params=par,
    )(l16, u2, v2)

    return out
